# baseline (speedup 1.0000x reference)
import functools

import jax
import jax.numpy as jnp
from jax import lax
from jax.experimental import pallas as pl
from jax.experimental.pallas import tpu as pltpu
from jax.experimental.pallas import tpu_sc as plsc

N_CORES = 2
N_SUB = 16
N_TILES = N_CORES * N_SUB
CHUNK = 128


def _round_up(v, m):
    return (v + m - 1) // m * m


def _sc_mesh():
    return plsc.VectorSubcoreMesh(core_axis_name="c", subcore_axis_name="s")


NBUF = 4


def _deg_body(n_chunks, stripe, dst_hbm, ones_hbm, zeros_hbm, out_hbm,
              ones_v, dsti_v, acc_sh, sem):
    cid = lax.axis_index("c")
    sid = lax.axis_index("s")
    w = cid * N_SUB + sid
    a_rows = stripe * N_SUB
    pltpu.sync_copy(ones_hbm, ones_v)
    pltpu.sync_copy(dst_hbm.at[pl.ds(w * n_chunks, n_chunks)], dsti_v)
    pltpu.sync_copy(zeros_hbm.at[pl.ds(sid * stripe, stripe)],
                    acc_sh.at[pl.ds(sid * stripe, stripe)])
    plsc.subcore_barrier()

    def body(i, carry):
        for b in range(NBUF):
            pltpu.async_copy(ones_v, acc_sh.at[dsti_v.at[i * NBUF + b]],
                             sem, add=True)
        for b in range(NBUF):
            pltpu.make_async_copy(ones_v, acc_sh.at[dsti_v.at[0]], sem).wait()
        return carry

    lax.fori_loop(0, n_chunks // NBUF, body, 0)
    plsc.subcore_barrier()
    pltpu.sync_copy(acc_sh.at[pl.ds(sid * stripe, stripe)],
                    out_hbm.at[pl.ds(cid * a_rows + sid * stripe, stripe)])


NB = 5
GA = 3


def _agg_body(n_chunks, stripe, dst_split, table_hbm, src_hbm, dst_hbm,
              zeros_hbm, out_hbm, srci_v, dsti_v,
              rows0, rows1, rows2, rows3, rows4,
              acc_sh, g0, g1, g2, g3, g4, s0, s1, s2, s3, s4):
    cid = lax.axis_index("c")
    sid = lax.axis_index("s")
    a_rows = stripe * N_SUB
    rows = (rows0, rows1, rows2, rows3, rows4)
    gs = (g0, g1, g2, g3, g4)
    ss = (s0, s1, s2, s3, s4)
    src_base = (cid * N_SUB + sid) * n_chunks
    dst_base = src_base if dst_split else sid * n_chunks
    pltpu.sync_copy(src_hbm.at[pl.ds(src_base, n_chunks)], srci_v)
    pltpu.sync_copy(dst_hbm.at[pl.ds(dst_base, n_chunks)], dsti_v)
    pltpu.sync_copy(zeros_hbm.at[pl.ds(sid * stripe, stripe)],
                    acc_sh.at[pl.ds(sid * stripe, stripe)])
    plsc.subcore_barrier()

    for c in range(GA):
        pltpu.async_copy(table_hbm.at[srci_v.at[c]], rows[c], gs[c])

    def body(i, carry):
        for b in range(NB):
            c = i * NB + b
            pltpu.make_async_copy(table_hbm.at[srci_v.at[0]],
                                  rows[b], gs[b]).wait()
            pltpu.async_copy(rows[b], acc_sh.at[dsti_v.at[c]], ss[b],
                             add=True)
            bn = (b + GA) % NB

            def _wait_prev_scatter():
                pltpu.make_async_copy(rows[bn], acc_sh.at[dsti_v.at[0]],
                                      ss[bn]).wait()

            if b + GA >= NB:
                _wait_prev_scatter()
            else:
                @pl.when(i > 0)
                def _():
                    _wait_prev_scatter()
            nxt = jnp.where(c + GA < n_chunks, c + GA, 0)
            pltpu.async_copy(table_hbm.at[srci_v.at[nxt]], rows[bn], gs[bn])
        return carry

    lax.fori_loop(0, n_chunks // NB, body, 0)
    for c in range(n_chunks - (NB - GA), n_chunks):
        b = c % NB
        pltpu.make_async_copy(rows[b], acc_sh.at[dsti_v.at[0]], ss[b]).wait()
    for c in range(n_chunks - GA, n_chunks):
        b = (c + GA) % NB
        pltpu.make_async_copy(table_hbm.at[srci_v.at[0]],
                              rows[b], gs[b]).wait()
    plsc.subcore_barrier()
    pltpu.sync_copy(acc_sh.at[pl.ds(sid * stripe, stripe)],
                    out_hbm.at[pl.ds(cid * a_rows + sid * stripe, stripe)])


def _make_deg_kernel(n_chunks, stripe):
    a_rows = stripe * N_SUB
    return pl.kernel(
        functools.partial(_deg_body, n_chunks, stripe),
        out_type=jax.ShapeDtypeStruct((N_CORES * a_rows, 16), jnp.float32),
        mesh=_sc_mesh(),
        compiler_params=pltpu.CompilerParams(use_tc_tiling_on_sc=False),
        scratch_types=[
            pltpu.VMEM((CHUNK, 16), jnp.float32),
            pltpu.VMEM((n_chunks, CHUNK), jnp.int32),
            pltpu.VMEM_SHARED((a_rows, 16), jnp.float32),
            pltpu.SemaphoreType.DMA,
        ],
    )


def _make_agg_kernel(n_chunks, stripe, d, dst_split=True):
    a_rows = stripe * N_SUB
    return pl.kernel(
        functools.partial(_agg_body, n_chunks, stripe, dst_split),
        out_type=jax.ShapeDtypeStruct((N_CORES * a_rows, d), jnp.float32),
        mesh=_sc_mesh(),
        compiler_params=pltpu.CompilerParams(use_tc_tiling_on_sc=(d % 128 == 0)),
        scratch_types=(
            [pltpu.VMEM((n_chunks, CHUNK), jnp.int32),
             pltpu.VMEM((n_chunks, CHUNK), jnp.int32)]
            + [pltpu.VMEM((CHUNK, d), jnp.float32) for _ in range(NB)]
            + [pltpu.VMEM_SHARED((a_rows, d), jnp.float32)]
            + [pltpu.SemaphoreType.DMA for _ in range(2 * NB)]
        ),
    )


def _tc_scale_body(x_ref, w1_ref, d0_ref, d1_ref, hs_ref, dinv_ref):
    deg = d0_ref[...] + d1_ref[...] + 1.0
    dinv = lax.rsqrt(deg)
    h = jnp.dot(x_ref[...], w1_ref[...], preferred_element_type=jnp.float32)
    hs = h * dinv
    n = hs.shape[0]
    half = hs.shape[1] // 2
    hs_ref[...] = jnp.concatenate([hs[:, :half], hs[:, half:]], axis=0)
    dinv_ref[...] = dinv


def _tc_mid_body(p0_ref, p1_ref, hsa_ref, hsb_ref, dinv_ref, b1_ref, w2_ref,
                 hs2_ref):
    dinv = dinv_ref[...]
    agg = jnp.concatenate([p0_ref[...], p1_ref[...]], axis=1)
    hs1 = jnp.concatenate([hsa_ref[...], hsb_ref[...]], axis=1)
    pre = (agg + hs1) * dinv + b1_ref[...]
    a1 = jnp.maximum(pre, 0.0)
    hs2_ref[...] = jnp.dot(a1, w2_ref[...],
                           preferred_element_type=jnp.float32) * dinv


def _tc_final_body(q0_ref, q1_ref, hs2_ref, dinv_ref, b2_ref, out_ref):
    out_ref[...] = ((q0_ref[...] + q1_ref[...] + hs2_ref[...])
                    * dinv_ref[...] + b2_ref[...])


def kernel(x, edge_index, W1, b1, W2, b2):
    n, in_ch = x.shape
    hid = W1.shape[1]
    out_ch = W2.shape[1]
    e = edge_index.shape[1]

    src = edge_index[0].astype(jnp.int32)
    dst = edge_index[1].astype(jnp.int32)

    e_pad = _round_up(e, N_TILES * CHUNK * NBUF * NB)
    n_chunks = e_pad // (N_TILES * CHUNK)
    src_p = jnp.concatenate(
        [src, jnp.zeros((e_pad - e,), jnp.int32)]).reshape(-1, CHUNK)
    dst_p = jnp.concatenate(
        [dst, jnp.full((e_pad - e,), n, jnp.int32)]).reshape(-1, CHUNK)

    a_rows = _round_up(n + 1, N_SUB * 8)
    stripe = a_rows // N_SUB
    d2 = 16

    half = hid // 2
    zeros1 = jnp.zeros((a_rows, half), jnp.float32)
    zeros2 = jnp.zeros((a_rows, d2), jnp.float32)
    ones16 = jnp.ones((CHUNK, 16), jnp.float32)

    src_fs = jnp.concatenate([src_p, src_p + n], axis=0)

    degp = _make_deg_kernel(n_chunks, stripe)(dst_p, ones16, zeros2)
    d0 = degp[:n, :1]
    d1 = degp[a_rows:a_rows + n, :1]

    hs_st, dinv = pl.pallas_call(
        _tc_scale_body,
        out_shape=[jax.ShapeDtypeStruct((2 * n, half), jnp.float32),
                   jax.ShapeDtypeStruct((n, 1), jnp.float32)],
    )(x, W1, d0, d1)

    p = _make_agg_kernel(2 * n_chunks, stripe, half, dst_split=False)(
        hs_st, src_fs, dst_p, zeros1)
    p0 = p[:n]
    p1 = p[a_rows:a_rows + n]

    w2p = jnp.zeros((hid, d2), jnp.float32).at[:, :out_ch].set(W2)
    b1r = b1.reshape(1, hid)
    hs2 = pl.pallas_call(
        _tc_mid_body,
        out_shape=jax.ShapeDtypeStruct((n, d2), jnp.float32),
    )(p0, p1, hs_st[:n], hs_st[n:], dinv, b1r, w2p)

    q = _make_agg_kernel(n_chunks, stripe, d2)(hs2, src_p, dst_p, zeros2)
    q0 = q[:n]
    q1 = q[a_rows:a_rows + n]

    b2p = jnp.zeros((1, d2), jnp.float32).at[0, :out_ch].set(b2)
    out16 = pl.pallas_call(
        _tc_final_body,
        out_shape=jax.ShapeDtypeStruct((n, d2), jnp.float32),
    )(q0, q1, hs2, dinv, b2p)

    return out16[:, :out_ch]


# speedup vs baseline: 1.6326x; 1.6326x over previous
import functools

import jax
import jax.numpy as jnp
from jax import lax
from jax.experimental import pallas as pl
from jax.experimental.pallas import tpu as pltpu
from jax.experimental.pallas import tpu_sc as plsc

N_CORES = 2
N_SUB = 16
N_TILES = N_CORES * N_SUB
CHUNK = 128
NBUF = 4
NB = 2


def _round_up(v, m):
    return (v + m - 1) // m * m


def _sc_mesh():
    return plsc.VectorSubcoreMesh(core_axis_name="c", subcore_axis_name="s")


def _deg_body(n_chunks, stripe, dst_hbm, ones_hbm, zeros_hbm, out_hbm,
              ones_v, dsti_v, acc_sh, sem):
    cid = lax.axis_index("c")
    sid = lax.axis_index("s")
    w = cid * N_SUB + sid
    a_rows = stripe * N_SUB
    pltpu.sync_copy(ones_hbm, ones_v)
    pltpu.sync_copy(dst_hbm.at[pl.ds(w * n_chunks, n_chunks)], dsti_v)
    pltpu.sync_copy(zeros_hbm.at[pl.ds(sid * stripe, stripe)],
                    acc_sh.at[pl.ds(sid * stripe, stripe)])
    plsc.subcore_barrier()

    def body(i, carry):
        for b in range(NBUF):
            pltpu.async_copy(ones_v, acc_sh.at[dsti_v.at[i * NBUF + b]],
                             sem, add=True)
        for b in range(NBUF):
            pltpu.make_async_copy(ones_v, acc_sh.at[dsti_v.at[0]], sem).wait()
        return carry

    lax.fori_loop(0, n_chunks // NBUF, body, 0)
    plsc.subcore_barrier()
    pltpu.sync_copy(acc_sh.at[pl.ds(sid * stripe, stripe)],
                    out_hbm.at[pl.ds(cid * a_rows + sid * stripe, stripe)])


def _agg_body(pc, stripe, t_rows, feat_split, table_hbm, pk_hbm, zeros_hbm,
              out_hbm, pki_v, si0, si1, di0, di1, rows0, rows1,
              tab_sh, acc_sh, g0, g1, s0, s1):
    cid = lax.axis_index("c")
    sid = lax.axis_index("s")
    a_rows = stripe * N_SUB
    t_stripe = t_rows // N_SUB
    rows = (rows0, rows1)
    sis = (si0, si1)
    dis = (di0, di1)
    gs = (g0, g1)
    ss = (s0, s1)
    pk_base = sid * pc if feat_split else (cid * N_SUB + sid) * pc
    t_base = cid * t_rows if feat_split else 0
    pltpu.sync_copy(pk_hbm.at[pl.ds(pk_base, pc)], pki_v)
    pltpu.sync_copy(table_hbm.at[pl.ds(t_base + sid * t_stripe, t_stripe)],
                    tab_sh.at[pl.ds(sid * t_stripe, t_stripe)])
    pltpu.sync_copy(zeros_hbm.at[pl.ds(sid * stripe, stripe)],
                    acc_sh.at[pl.ds(sid * stripe, stripe)])
    plsc.subcore_barrier()

    def unpack(c, b):
        for g in range(CHUNK // 16):
            wv = pki_v[c, pl.ds(g * 16, 16)]
            sis[b][pl.ds(g * 16, 16)] = wv & 0xFFFF
            dis[b][pl.ds(g * 16, 16)] = wv >> 16

    unpack(0, 0)
    pltpu.async_copy(tab_sh.at[sis[0]], rows[0], gs[0])

    def body(i, carry):
        for b in range(NB):
            c = i * NB + b
            bn = (b + 1) % NB
            cn = jnp.where(c + 1 < pc, c + 1, 0)

            def _wait_sc():
                pltpu.make_async_copy(rows[bn], acc_sh.at[dis[bn]],
                                      ss[bn]).wait()
            if b == 0:
                @pl.when(i > 0)
                def _():
                    _wait_sc()
            else:
                _wait_sc()
            unpack(cn, bn)
            pltpu.async_copy(tab_sh.at[sis[bn]], rows[bn], gs[bn])
            pltpu.make_async_copy(tab_sh.at[sis[0]], rows[b], gs[b]).wait()
            pltpu.async_copy(rows[b], acc_sh.at[dis[b]], ss[b], add=True)
        return carry

    lax.fori_loop(0, pc // NB, body, 0)
    pltpu.make_async_copy(rows[(pc - 1) % NB], acc_sh.at[dis[0]],
                          ss[(pc - 1) % NB]).wait()
    pltpu.make_async_copy(tab_sh.at[sis[0]], rows[pc % NB], gs[pc % NB]).wait()
    plsc.subcore_barrier()
    pltpu.sync_copy(acc_sh.at[pl.ds(sid * stripe, stripe)],
                    out_hbm.at[pl.ds(cid * a_rows + sid * stripe, stripe)])


def _make_deg_kernel(n_chunks, stripe):
    a_rows = stripe * N_SUB
    return pl.kernel(
        functools.partial(_deg_body, n_chunks, stripe),
        out_type=jax.ShapeDtypeStruct((N_CORES * a_rows, 16), jnp.float32),
        mesh=_sc_mesh(),
        compiler_params=pltpu.CompilerParams(use_tc_tiling_on_sc=False),
        scratch_types=[
            pltpu.VMEM((CHUNK, 16), jnp.float32),
            pltpu.VMEM((n_chunks, CHUNK), jnp.int32),
            pltpu.VMEM_SHARED((a_rows, 16), jnp.float32),
            pltpu.SemaphoreType.DMA,
        ],
    )


def _make_agg_kernel(pc, stripe, t_rows, d, feat_split):
    a_rows = stripe * N_SUB
    return pl.kernel(
        functools.partial(_agg_body, pc, stripe, t_rows, feat_split),
        out_type=jax.ShapeDtypeStruct((N_CORES * a_rows, d), jnp.float32),
        mesh=_sc_mesh(),
        compiler_params=pltpu.CompilerParams(use_tc_tiling_on_sc=False),
        scratch_types=(
            [pltpu.VMEM((pc, CHUNK), jnp.int32)]
            + [pltpu.VMEM((CHUNK,), jnp.int32) for _ in range(4)]
            + [pltpu.VMEM((CHUNK, d), jnp.float32) for _ in range(NB)]
            + [pltpu.VMEM_SHARED((t_rows, d), jnp.float32),
               pltpu.VMEM_SHARED((a_rows, d), jnp.float32)]
            + [pltpu.SemaphoreType.DMA for _ in range(2 * NB)]
        ),
    )


def _tc_scale_body(x_ref, w1_ref, d0_ref, d1_ref, hs_ref, dinv_ref):
    deg = d0_ref[...] + d1_ref[...] + 1.0
    dinv = lax.rsqrt(deg)
    h = jnp.dot(x_ref[...], w1_ref[...], preferred_element_type=jnp.float32)
    hs = h * dinv
    half = hs.shape[1] // 2
    hs_ref[...] = jnp.concatenate([hs[:, :half], hs[:, half:]], axis=0)
    dinv_ref[...] = dinv


def _tc_mid_body(p0_ref, p1_ref, hsa_ref, hsb_ref, dinv_ref, b1_ref, w2_ref,
                 hs2_ref):
    dinv = dinv_ref[...]
    agg = jnp.concatenate([p0_ref[...], p1_ref[...]], axis=1)
    hs1 = jnp.concatenate([hsa_ref[...], hsb_ref[...]], axis=1)
    pre = (agg + hs1) * dinv + b1_ref[...]
    a1 = jnp.maximum(pre, 0.0)
    hs2_ref[...] = jnp.dot(a1, w2_ref[...],
                           preferred_element_type=jnp.float32) * dinv


def _tc_final_body(q0_ref, q1_ref, hs2_ref, dinv_ref, b2_ref, out_ref):
    out_ref[...] = ((q0_ref[...] + q1_ref[...] + hs2_ref[...])
                    * dinv_ref[...] + b2_ref[...])


def kernel(x, edge_index, W1, b1, W2, b2):
    n, in_ch = x.shape
    hid = W1.shape[1]
    out_ch = W2.shape[1]
    e = edge_index.shape[1]

    src = edge_index[0].astype(jnp.int32)
    dst = edge_index[1].astype(jnp.int32)

    e_pad = _round_up(e, N_TILES * CHUNK * NBUF)
    n_chunks = e_pad // (N_TILES * CHUNK)
    pc1 = N_CORES * n_chunks
    dst_p = jnp.concatenate(
        [dst, jnp.full((e_pad - e,), n, jnp.int32)]).reshape(-1, CHUNK)
    packed = jnp.concatenate(
        [src | (dst << 16),
         jnp.full((e_pad - e,), n << 16, jnp.int32)]).reshape(-1, CHUNK)

    a_rows = _round_up(n + 1, N_SUB * 8)
    stripe = a_rows // N_SUB
    half = hid // 2
    d2 = 16

    zeros1 = jnp.zeros((a_rows, half), jnp.float32)
    zeros2 = jnp.zeros((a_rows, d2), jnp.float32)
    ones16 = jnp.ones((CHUNK, 16), jnp.float32)

    degp = _make_deg_kernel(n_chunks, stripe)(dst_p, ones16, zeros2)
    d0 = degp[:n, :1]
    d1 = degp[a_rows:a_rows + n, :1]

    hs_st, dinv = pl.pallas_call(
        _tc_scale_body,
        out_shape=[jax.ShapeDtypeStruct((2 * n, half), jnp.float32),
                   jax.ShapeDtypeStruct((n, 1), jnp.float32)],
    )(x, W1, d0, d1)

    p = _make_agg_kernel(pc1, stripe, n, half, True)(hs_st, packed, zeros1)
    p0 = p[:n]
    p1 = p[a_rows:a_rows + n]

    w2p = jnp.zeros((hid, d2), jnp.float32).at[:, :out_ch].set(W2)
    b1r = b1.reshape(1, hid)
    hs2 = pl.pallas_call(
        _tc_mid_body,
        out_shape=jax.ShapeDtypeStruct((n, d2), jnp.float32),
    )(p0, p1, hs_st[:n], hs_st[n:], dinv, b1r, w2p)

    q = _make_agg_kernel(n_chunks, stripe, n, d2, False)(hs2, packed, zeros2)
    q0 = q[:n]
    q1 = q[a_rows:a_rows + n]

    b2p = jnp.zeros((1, d2), jnp.float32).at[0, :out_ch].set(b2)
    out16 = pl.pallas_call(
        _tc_final_body,
        out_shape=jax.ShapeDtypeStruct((n, d2), jnp.float32),
    )(q0, q1, hs2, dinv, b2p)

    return out16[:, :out_ch]
